# baseline (speedup 1.0000x reference)
import jax
import jax.numpy as jnp
from jax import lax
from jax.experimental import pallas as pl
from jax.experimental.pallas import tpu as pltpu
from jax.experimental.pallas import tpu_sc as plsc

N = 10000
E = 320000
D = 128
FF = 256
NC = 2
NS = 16
NW = NC * NS
EW = E // NW
K = 100
NCH = EW // K
NB = 10
NBL = NCH // NB
NP = 10112
RS = NP // NS
CW = 16
DA = D + CW

_mesh = plsc.VectorSubcoreMesh(
    core_axis_name="c", subcore_axis_name="s", num_cores=NC, num_subcores=NS)
_sc_params = pltpu.CompilerParams(use_tc_tiling_on_sc=False)


def _make_sc_kernel(width):
    def body(h_hbm, src_hbm, dst_hbm, zs_hbm, s_out,
             sbuf0, sbuf1, dbuf0, dbuf1, rows0, rows1,
             isem0, isem1, gsem0, gsem1, ssem0, ssem1, shared_s):
        cid = lax.axis_index("c")
        sid = lax.axis_index("s")
        sbufs = (sbuf0, sbuf1)
        dbufs = (dbuf0, dbuf1)
        rows = (rows0, rows1)
        isems = (isem0, isem1)
        gsems = (gsem0, gsem1)
        ssems = (ssem0, ssem1)

        pltpu.sync_copy(zs_hbm, shared_s.at[pl.ds(sid * RS, RS)])
        plsc.subcore_barrier()

        def stage_start(blk, b):
            pltpu.async_copy(src_hbm.at[cid, sid, blk], sbufs[b], isems[b])
            pltpu.async_copy(dst_hbm.at[cid, sid, blk], dbufs[b], isems[b])

        def stage_wait(blk, b):
            pltpu.make_async_copy(src_hbm.at[cid, sid, blk], sbufs[b],
                                  isems[b]).wait()
            pltpu.make_async_copy(dst_hbm.at[cid, sid, blk], dbufs[b],
                                  isems[b]).wait()

        def gather_start(r, b, t):
            pltpu.async_copy(h_hbm.at[sbufs[b].at[t]], rows[r], gsems[r])

        def gather_wait(r, b, t):
            pltpu.make_async_copy(h_hbm.at[sbufs[b].at[t]], rows[r],
                                  gsems[r]).wait()

        def scat_start(r, b, t):
            pltpu.async_copy(rows[r], shared_s.at[dbufs[b].at[t]], ssems[r],
                             add=True)

        def scat_wait(r, b, t):
            pltpu.make_async_copy(rows[r], shared_s.at[dbufs[b].at[t]],
                                  ssems[r]).wait()

        def emit_block(cur, nxt, blk, stage_next, first_block, has_next):
            for t in range(NB):
                last = t == NB - 1
                gather_wait(t % 2, cur, t)
                if t > 0:
                    scat_wait((t + 1) % 2, cur, t - 1)
                elif not first_block:
                    scat_wait(1, nxt, NB - 1)
                if t == 0 and stage_next:
                    stage_start(blk + 1, nxt)
                if not last:
                    gather_start((t + 1) % 2, cur, t + 1)
                elif has_next:
                    stage_wait(blk + 1, nxt)
                    gather_start(0, nxt, 0)
                scat_start(t % 2, cur, t)

        stage_start(0, 0)
        stage_wait(0, 0)
        gather_start(0, 0, 0)
        emit_block(0, 1, 0, True, True, True)

        def loop(bp, _):
            b = 2 * bp + 1
            emit_block(1, 0, b, True, False, True)
            emit_block(0, 1, b + 1, True, False, True)
            return 0

        lax.fori_loop(0, (NBL - 2) // 2, loop, 0)
        emit_block(1, 0, NBL - 1, False, False, False)
        scat_wait(1, 1, NB - 1)

        plsc.subcore_barrier()
        pltpu.sync_copy(shared_s.at[pl.ds(sid * RS, RS)],
                        s_out.at[cid, pl.ds(sid * RS, RS)])

    return pl.kernel(
        body,
        out_type=jax.ShapeDtypeStruct((NC, NP, width), jnp.float32),
        mesh=_mesh,
        compiler_params=_sc_params,
        scratch_types=[
            pltpu.VMEM((NB, K), jnp.int32),
            pltpu.VMEM((NB, K), jnp.int32),
            pltpu.VMEM((NB, K), jnp.int32),
            pltpu.VMEM((NB, K), jnp.int32),
            pltpu.VMEM((K, width), jnp.float32),
            pltpu.VMEM((K, width), jnp.float32),
            pltpu.SemaphoreType.DMA,
            pltpu.SemaphoreType.DMA,
            pltpu.SemaphoreType.DMA,
            pltpu.SemaphoreType.DMA,
            pltpu.SemaphoreType.DMA,
            pltpu.SemaphoreType.DMA,
            pltpu.VMEM_SHARED((NP, width), jnp.float32),
        ])


_sc_seg_aug = _make_sc_kernel(DA)
_sc_seg = _make_sc_kernel(D)


def _ln_block(t, g, b):
    m = jnp.mean(t, axis=-1, keepdims=True)
    d = t - m
    v = jnp.mean(d * d, axis=-1, keepdims=True)
    return d * lax.rsqrt(v + 1e-5) * g + b


def _dot(a, b):
    return jnp.dot(a, b, preferred_element_type=jnp.float32)


def _dense(h, mean, ws, wn, bg, w1, b1, w2, b2, g1, be1, g2, be2):
    x2 = _dot(h, ws) + _dot(mean, wn) + bg
    t = _ln_block(h + x2, g1, be1)
    ff = _dot(jnp.maximum(_dot(t, w1) + b1, 0.0), w2) + b2
    return _ln_block(t + ff, g2, be2)


def _tc1_body(h_ref, s_ref, ws_ref, wn_ref, bg_ref, w1_ref, b1_ref,
              w2_ref, b2_ref, g1_ref, be1_ref, g2_ref, be2_ref,
              out_ref, inv_ref):
    v = s_ref[0] + s_ref[1]
    cnt = jnp.max(v[:, D:], axis=-1, keepdims=True)
    inv = 1.0 / jnp.maximum(cnt, 1.0)
    mean = v[:, :D] * inv
    out_ref[...] = _dense(h_ref[...], mean, ws_ref[...], wn_ref[...],
                          bg_ref[...], w1_ref[...], b1_ref[...], w2_ref[...],
                          b2_ref[...], g1_ref[...], be1_ref[...],
                          g2_ref[...], be2_ref[...])
    inv_ref[...] = jnp.broadcast_to(inv, inv_ref.shape)


def _tc2_body(h_ref, s_ref, inv_ref, ws_ref, wn_ref, bg_ref, w1_ref, b1_ref,
              w2_ref, b2_ref, g1_ref, be1_ref, g2_ref, be2_ref, out_ref):
    mean = (s_ref[0] + s_ref[1]) * inv_ref[:, :1]
    out_ref[...] = _dense(h_ref[...], mean, ws_ref[...], wn_ref[...],
                          bg_ref[...], w1_ref[...], b1_ref[...], w2_ref[...],
                          b2_ref[...], g1_ref[...], be1_ref[...],
                          g2_ref[...], be2_ref[...])


_TC_BLK = 2000
_IW = 8


def _wspecs():
    full = lambda shape: pl.BlockSpec(shape, lambda i: (0,) * len(shape))
    return [full((D, D)), full((D, D)), full((1, D)),
            full((D, FF)), full((1, FF)), full((FF, D)), full((1, D)),
            full((1, D)), full((1, D)), full((1, D)), full((1, D))]


def _tc_dense1(h, s, *weights):
    return pl.pallas_call(
        _tc1_body,
        grid=(N // _TC_BLK,),
        in_specs=[
            pl.BlockSpec((_TC_BLK, D), lambda i: (i, 0)),
            pl.BlockSpec((NC, _TC_BLK, DA), lambda i: (0, i, 0)),
        ] + _wspecs(),
        out_specs=[pl.BlockSpec((_TC_BLK, D), lambda i: (i, 0)),
                   pl.BlockSpec((_TC_BLK, _IW), lambda i: (i, 0))],
        out_shape=[jax.ShapeDtypeStruct((N, D), jnp.float32),
                   jax.ShapeDtypeStruct((N, _IW), jnp.float32)],
    )(h, s, *weights)


def _tc_dense2(h, s, inv, *weights):
    return pl.pallas_call(
        _tc2_body,
        grid=(N // _TC_BLK,),
        in_specs=[
            pl.BlockSpec((_TC_BLK, D), lambda i: (i, 0)),
            pl.BlockSpec((NC, _TC_BLK, D), lambda i: (0, i, 0)),
            pl.BlockSpec((_TC_BLK, _IW), lambda i: (i, 0)),
        ] + _wspecs(),
        out_specs=pl.BlockSpec((_TC_BLK, D), lambda i: (i, 0)),
        out_shape=jax.ShapeDtypeStruct((N, D), jnp.float32),
    )(h, s, inv, *weights)


def kernel(x, edge_index, W_self, W_nbr, b_gnn, W1, b1, W2, b2,
           ln1_g, ln1_b, ln2_g, ln2_b):
    src = edge_index[0].reshape(NC, NS, NBL, NB, K)
    dst = edge_index[1].reshape(NC, NS, NBL, NB, K)
    zs_aug = jnp.zeros((RS, DA), jnp.float32)
    zs = jnp.zeros((RS, D), jnp.float32)
    x_aug = jnp.concatenate([x, jnp.ones((N, CW), jnp.float32)], axis=1)

    r2 = lambda a: a.reshape(1, -1)
    w = lambda i: (W_self[i], W_nbr[i], r2(b_gnn[i]), W1[i], r2(b1[i]),
                   W2[i], r2(b2[i]), r2(ln1_g[i]), r2(ln1_b[i]),
                   r2(ln2_g[i]), r2(ln2_b[i]))

    s_parts = _sc_seg_aug(x_aug, src, dst, zs_aug)
    h, inv = _tc_dense1(x, s_parts, *w(0))
    s_parts = _sc_seg(h, src, dst, zs)
    h = _tc_dense2(h, s_parts, inv, *w(1))
    return h


# speedup vs baseline: 1.2522x; 1.2522x over previous
import jax
import jax.numpy as jnp
from jax import lax
from jax.experimental import pallas as pl
from jax.experimental.pallas import tpu as pltpu
from jax.experimental.pallas import tpu_sc as plsc

N = 10000
E = 320000
D = 128
FF = 256
NC = 2
NS = 16
NW = NC * NS
EW = E // NW
K = 50
NCH = EW // K
NB = 20
NBL = NCH // NB
NR = 4
NP = 10112
RS = NP // NS
CW = 16
DA = D + CW

_mesh = plsc.VectorSubcoreMesh(
    core_axis_name="c", subcore_axis_name="s", num_cores=NC, num_subcores=NS)
_sc_params = pltpu.CompilerParams(use_tc_tiling_on_sc=False)


def _make_sc_kernel(width):
    def body(h_hbm, src_hbm, dst_hbm, zs_hbm, s_out,
             sbuf0, sbuf1, dbuf0, dbuf1, rows_all, isem0, isem1,
             gsem0, gsem1, gsem2, gsem3, ssem0, ssem1, ssem2, ssem3,
             shared_s):
        cid = lax.axis_index("c")
        sid = lax.axis_index("s")
        sbufs = (sbuf0, sbuf1)
        dbufs = (dbuf0, dbuf1)
        isems = (isem0, isem1)
        gsems = (gsem0, gsem1, gsem2, gsem3)
        ssems = (ssem0, ssem1, ssem2, ssem3)

        pltpu.sync_copy(zs_hbm, shared_s.at[pl.ds(sid * RS, RS)])
        plsc.subcore_barrier()

        def stage_start(blk, b):
            pltpu.async_copy(src_hbm.at[cid, sid, blk], sbufs[b], isems[b])
            pltpu.async_copy(dst_hbm.at[cid, sid, blk], dbufs[b], isems[b])

        def stage_wait(blk, b):
            pltpu.make_async_copy(src_hbm.at[cid, sid, blk], sbufs[b],
                                  isems[b]).wait()
            pltpu.make_async_copy(dst_hbm.at[cid, sid, blk], dbufs[b],
                                  isems[b]).wait()

        def gather_start(r, b, t):
            pltpu.async_copy(h_hbm.at[sbufs[b].at[t]], rows_all.at[r],
                             gsems[r])

        def gather_wait(r, b, t):
            pltpu.make_async_copy(h_hbm.at[sbufs[b].at[t]], rows_all.at[r],
                                  gsems[r]).wait()

        def scat_start(r, b, t):
            pltpu.async_copy(rows_all.at[r], shared_s.at[dbufs[b].at[t]],
                             ssems[r], add=True)

        def scat_wait(r, b, t):
            pltpu.make_async_copy(rows_all.at[r], shared_s.at[dbufs[b].at[t]],
                                  ssems[r]).wait()

        def emit_block(cur, nxt, blk, stage_next, first, has_next):
            for t in range(NB):
                gather_wait(t % NR, cur, t)
                if t > 0:
                    scat_wait((t - 1) % NR, cur, t - 1)
                elif not first:
                    scat_wait((NB - 1) % NR, nxt, NB - 1)
                if t == 0 and stage_next:
                    stage_start(blk + 1, nxt)
                if t < NB - 3:
                    gather_start((t + 3) % NR, cur, t + 3)
                elif has_next:
                    if t == NB - 3:
                        stage_wait(blk + 1, nxt)
                    gather_start((t + 3) % NR, nxt, t + 3 - NB)
                scat_start(t % NR, cur, t)

        stage_start(0, 0)
        stage_wait(0, 0)
        gather_start(0, 0, 0)
        gather_start(1, 0, 1)
        gather_start(2, 0, 2)
        emit_block(0, 1, 0, True, True, True)

        def loop(bp, _):
            b = 2 * bp + 1
            emit_block(1, 0, b, True, False, True)
            emit_block(0, 1, b + 1, True, False, True)
            return 0

        lax.fori_loop(0, (NBL - 2) // 2, loop, 0)
        emit_block(1, 0, NBL - 1, False, False, False)
        scat_wait((NB - 1) % NR, 1, NB - 1)

        plsc.subcore_barrier()
        pltpu.sync_copy(shared_s.at[pl.ds(sid * RS, RS)],
                        s_out.at[cid, pl.ds(sid * RS, RS)])

    return pl.kernel(
        body,
        out_type=jax.ShapeDtypeStruct((NC, NP, width), jnp.float32),
        mesh=_mesh,
        compiler_params=_sc_params,
        scratch_types=[
            pltpu.VMEM((NB, K), jnp.int32),
            pltpu.VMEM((NB, K), jnp.int32),
            pltpu.VMEM((NB, K), jnp.int32),
            pltpu.VMEM((NB, K), jnp.int32),
            pltpu.VMEM((NR, K, width), jnp.float32),
            pltpu.SemaphoreType.DMA,
            pltpu.SemaphoreType.DMA,
            pltpu.SemaphoreType.DMA,
            pltpu.SemaphoreType.DMA,
            pltpu.SemaphoreType.DMA,
            pltpu.SemaphoreType.DMA,
            pltpu.SemaphoreType.DMA,
            pltpu.SemaphoreType.DMA,
            pltpu.SemaphoreType.DMA,
            pltpu.SemaphoreType.DMA,
            pltpu.VMEM_SHARED((NP, width), jnp.float32),
        ])


_sc_seg_aug = _make_sc_kernel(DA)
_sc_seg = _make_sc_kernel(D)


def _ln_block(t, g, b):
    m = jnp.mean(t, axis=-1, keepdims=True)
    d = t - m
    v = jnp.mean(d * d, axis=-1, keepdims=True)
    return d * lax.rsqrt(v + 1e-5) * g + b


def _dot(a, b):
    return jnp.dot(a, b, preferred_element_type=jnp.float32)


def _dense(h, mean, ws, wn, bg, w1, b1, w2, b2, g1, be1, g2, be2):
    x2 = _dot(h, ws) + _dot(mean, wn) + bg
    t = _ln_block(h + x2, g1, be1)
    ff = _dot(jnp.maximum(_dot(t, w1) + b1, 0.0), w2) + b2
    return _ln_block(t + ff, g2, be2)


def _tc1_body(h_ref, s_ref, ws_ref, wn_ref, bg_ref, w1_ref, b1_ref,
              w2_ref, b2_ref, g1_ref, be1_ref, g2_ref, be2_ref,
              out_ref, inv_ref):
    v = s_ref[0] + s_ref[1]
    cnt = jnp.max(v[:, D:], axis=-1, keepdims=True)
    inv = 1.0 / jnp.maximum(cnt, 1.0)
    mean = v[:, :D] * inv
    out_ref[...] = _dense(h_ref[...], mean, ws_ref[...], wn_ref[...],
                          bg_ref[...], w1_ref[...], b1_ref[...], w2_ref[...],
                          b2_ref[...], g1_ref[...], be1_ref[...],
                          g2_ref[...], be2_ref[...])
    inv_ref[...] = jnp.broadcast_to(inv, inv_ref.shape)


def _tc2_body(h_ref, s_ref, inv_ref, ws_ref, wn_ref, bg_ref, w1_ref, b1_ref,
              w2_ref, b2_ref, g1_ref, be1_ref, g2_ref, be2_ref, out_ref):
    mean = (s_ref[0] + s_ref[1]) * inv_ref[:, :1]
    out_ref[...] = _dense(h_ref[...], mean, ws_ref[...], wn_ref[...],
                          bg_ref[...], w1_ref[...], b1_ref[...], w2_ref[...],
                          b2_ref[...], g1_ref[...], be1_ref[...],
                          g2_ref[...], be2_ref[...])


_TC_BLK = 2000
_IW = 8


def _wspecs():
    full = lambda shape: pl.BlockSpec(shape, lambda i: (0,) * len(shape))
    return [full((D, D)), full((D, D)), full((1, D)),
            full((D, FF)), full((1, FF)), full((FF, D)), full((1, D)),
            full((1, D)), full((1, D)), full((1, D)), full((1, D))]


def _tc_dense1(h, s, *weights):
    return pl.pallas_call(
        _tc1_body,
        grid=(N // _TC_BLK,),
        in_specs=[
            pl.BlockSpec((_TC_BLK, D), lambda i: (i, 0)),
            pl.BlockSpec((NC, _TC_BLK, DA), lambda i: (0, i, 0)),
        ] + _wspecs(),
        out_specs=[pl.BlockSpec((_TC_BLK, D), lambda i: (i, 0)),
                   pl.BlockSpec((_TC_BLK, _IW), lambda i: (i, 0))],
        out_shape=[jax.ShapeDtypeStruct((N, D), jnp.float32),
                   jax.ShapeDtypeStruct((N, _IW), jnp.float32)],
    )(h, s, *weights)


def _tc_dense2(h, s, inv, *weights):
    return pl.pallas_call(
        _tc2_body,
        grid=(N // _TC_BLK,),
        in_specs=[
            pl.BlockSpec((_TC_BLK, D), lambda i: (i, 0)),
            pl.BlockSpec((NC, _TC_BLK, D), lambda i: (0, i, 0)),
            pl.BlockSpec((_TC_BLK, _IW), lambda i: (i, 0)),
        ] + _wspecs(),
        out_specs=pl.BlockSpec((_TC_BLK, D), lambda i: (i, 0)),
        out_shape=jax.ShapeDtypeStruct((N, D), jnp.float32),
    )(h, s, inv, *weights)


def kernel(x, edge_index, W_self, W_nbr, b_gnn, W1, b1, W2, b2,
           ln1_g, ln1_b, ln2_g, ln2_b):
    src = edge_index[0].reshape(NC, NS, NBL, NB, K)
    dst = edge_index[1].reshape(NC, NS, NBL, NB, K)
    zs_aug = jnp.zeros((RS, DA), jnp.float32)
    zs = jnp.zeros((RS, D), jnp.float32)
    x_aug = jnp.concatenate([x, jnp.ones((N, CW), jnp.float32)], axis=1)

    r2 = lambda a: a.reshape(1, -1)
    w = lambda i: (W_self[i], W_nbr[i], r2(b_gnn[i]), W1[i], r2(b1[i]),
                   W2[i], r2(b2[i]), r2(ln1_g[i]), r2(ln1_b[i]),
                   r2(ln2_g[i]), r2(ln2_b[i]))

    s_parts = _sc_seg_aug(x_aug, src, dst, zs_aug)
    h, inv = _tc_dense1(x, s_parts, *w(0))
    s_parts = _sc_seg(h, src, dst, zs)
    h = _tc_dense2(h, s_parts, inv, *w(1))
    return h


# speedup vs baseline: 1.2535x; 1.0011x over previous
import jax
import jax.numpy as jnp
from jax import lax
from jax.experimental import pallas as pl
from jax.experimental.pallas import tpu as pltpu
from jax.experimental.pallas import tpu_sc as plsc

N = 10000
E = 320000
D = 128
FF = 256
NC = 2
NS = 16
NW = NC * NS
EW = E // NW
K = 50
NCH = EW // K
NB = 20
NBL = NCH // NB
NR = 4
NP = 10112
RS = NP // NS
CW = 16
DA = D + CW

_sc_params = pltpu.CompilerParams(use_tc_tiling_on_sc=False)


def _make_sc_kernel(width):
    def body(h_hbm, src_hbm, dst_hbm, zs_hbm, s_out,
             sbuf0, sbuf1, dbuf0, dbuf1, rows_all, isem0, isem1,
             gsem0, gsem1, gsem2, gsem3, ssem0, ssem1, ssem2, ssem3,
             shared_s):
        cid = lax.axis_index("c")
        sid = lax.axis_index("s")
        sbufs = (sbuf0, sbuf1)
        dbufs = (dbuf0, dbuf1)
        isems = (isem0, isem1)
        gsems = (gsem0, gsem1, gsem2, gsem3)
        ssems = (ssem0, ssem1, ssem2, ssem3)

        pltpu.sync_copy(zs_hbm, shared_s.at[pl.ds(sid * RS, RS)])
        plsc.subcore_barrier()

        def stage_start(blk, b):
            pltpu.async_copy(src_hbm.at[cid, sid, blk], sbufs[b], isems[b])
            pltpu.async_copy(dst_hbm.at[cid, sid, blk], dbufs[b], isems[b])

        def stage_wait(blk, b):
            pltpu.make_async_copy(src_hbm.at[cid, sid, blk], sbufs[b],
                                  isems[b]).wait()
            pltpu.make_async_copy(dst_hbm.at[cid, sid, blk], dbufs[b],
                                  isems[b]).wait()

        def gather_start(r, b, t):
            pltpu.async_copy(h_hbm.at[sbufs[b].at[t]], rows_all.at[r],
                             gsems[r])

        def gather_wait(r, b, t):
            pltpu.make_async_copy(h_hbm.at[sbufs[b].at[t]], rows_all.at[r],
                                  gsems[r]).wait()

        def scat_start(r, b, t):
            pltpu.async_copy(rows_all.at[r], shared_s.at[dbufs[b].at[t]],
                             ssems[r], add=True)

        def scat_wait(r, b, t):
            pltpu.make_async_copy(rows_all.at[r], shared_s.at[dbufs[b].at[t]],
                                  ssems[r]).wait()

        def emit_block(cur, nxt, blk, stage_next, first, has_next):
            for t in range(NB):
                gather_wait(t % NR, cur, t)
                if t > 0:
                    scat_wait((t - 1) % NR, cur, t - 1)
                elif not first:
                    scat_wait((NB - 1) % NR, nxt, NB - 1)
                if t == 0 and stage_next:
                    stage_start(blk + 1, nxt)
                if t < NB - 3:
                    gather_start((t + 3) % NR, cur, t + 3)
                elif has_next:
                    if t == NB - 3:
                        stage_wait(blk + 1, nxt)
                    gather_start((t + 3) % NR, nxt, t + 3 - NB)
                scat_start(t % NR, cur, t)

        stage_start(0, 0)
        stage_wait(0, 0)
        gather_start(0, 0, 0)
        gather_start(1, 0, 1)
        gather_start(2, 0, 2)
        emit_block(0, 1, 0, True, True, True)

        def loop(bp, _):
            b = 2 * bp + 1
            emit_block(1, 0, b, True, False, True)
            emit_block(0, 1, b + 1, True, False, True)
            return 0

        lax.fori_loop(0, (NBL - 2) // 2, loop, 0)
        emit_block(1, 0, NBL - 1, False, False, False)
        scat_wait((NB - 1) % NR, 1, NB - 1)

        plsc.subcore_barrier()
        pltpu.sync_copy(shared_s.at[pl.ds(sid * RS, RS)],
                        s_out.at[cid, pl.ds(sid * RS, RS)])

    mesh = plsc.VectorSubcoreMesh(
        core_axis_name="c", subcore_axis_name="s",
        num_cores=NC, num_subcores=NS)
    return pl.kernel(
        body,
        out_type=jax.ShapeDtypeStruct((NC, NP, width), jnp.float32),
        mesh=mesh,
        compiler_params=_sc_params,
        scratch_types=[
            pltpu.VMEM((NB, K), jnp.int32),
            pltpu.VMEM((NB, K), jnp.int32),
            pltpu.VMEM((NB, K), jnp.int32),
            pltpu.VMEM((NB, K), jnp.int32),
            pltpu.VMEM((NR, K, width), jnp.float32),
            pltpu.SemaphoreType.DMA,
            pltpu.SemaphoreType.DMA,
            pltpu.SemaphoreType.DMA,
            pltpu.SemaphoreType.DMA,
            pltpu.SemaphoreType.DMA,
            pltpu.SemaphoreType.DMA,
            pltpu.SemaphoreType.DMA,
            pltpu.SemaphoreType.DMA,
            pltpu.SemaphoreType.DMA,
            pltpu.SemaphoreType.DMA,
            pltpu.VMEM_SHARED((NP, width), jnp.float32),
        ])


_sc_kernels = {}


def _get_sc_kernel(width):
    if width not in _sc_kernels:
        _sc_kernels[width] = _make_sc_kernel(width)
    return _sc_kernels[width]


def _ln_block(t, g, b):
    m = jnp.mean(t, axis=-1, keepdims=True)
    d = t - m
    v = jnp.mean(d * d, axis=-1, keepdims=True)
    return d * lax.rsqrt(v + 1e-5) * g + b


def _dot(a, b):
    return jnp.dot(a, b, preferred_element_type=jnp.float32)


def _dense(h, mean, ws, wn, bg, w1, b1, w2, b2, g1, be1, g2, be2):
    x2 = _dot(h, ws) + _dot(mean, wn) + bg
    t = _ln_block(h + x2, g1, be1)
    ff = _dot(jnp.maximum(_dot(t, w1) + b1, 0.0), w2) + b2
    return _ln_block(t + ff, g2, be2)


def _tc1_body(h_ref, s_ref, ws_ref, wn_ref, bg_ref, w1_ref, b1_ref,
              w2_ref, b2_ref, g1_ref, be1_ref, g2_ref, be2_ref,
              out_ref, inv_ref):
    v = s_ref[0] + s_ref[1]
    cnt = jnp.max(v[:, D:], axis=-1, keepdims=True)
    inv = 1.0 / jnp.maximum(cnt, 1.0)
    mean = v[:, :D] * inv
    out_ref[...] = _dense(h_ref[...], mean, ws_ref[...], wn_ref[...],
                          bg_ref[...], w1_ref[...], b1_ref[...], w2_ref[...],
                          b2_ref[...], g1_ref[...], be1_ref[...],
                          g2_ref[...], be2_ref[...])
    inv_ref[...] = jnp.broadcast_to(inv, inv_ref.shape)


def _tc2_body(h_ref, s_ref, inv_ref, ws_ref, wn_ref, bg_ref, w1_ref, b1_ref,
              w2_ref, b2_ref, g1_ref, be1_ref, g2_ref, be2_ref, out_ref):
    mean = (s_ref[0] + s_ref[1]) * inv_ref[:, :1]
    out_ref[...] = _dense(h_ref[...], mean, ws_ref[...], wn_ref[...],
                          bg_ref[...], w1_ref[...], b1_ref[...], w2_ref[...],
                          b2_ref[...], g1_ref[...], be1_ref[...],
                          g2_ref[...], be2_ref[...])


_TC_BLK = 2000
_IW = 8


def _wspecs():
    full = lambda shape: pl.BlockSpec(shape, lambda i: (0,) * len(shape))
    return [full((D, D)), full((D, D)), full((1, D)),
            full((D, FF)), full((1, FF)), full((FF, D)), full((1, D)),
            full((1, D)), full((1, D)), full((1, D)), full((1, D))]


def _tc_dense1(h, s, *weights):
    return pl.pallas_call(
        _tc1_body,
        grid=(N // _TC_BLK,),
        in_specs=[
            pl.BlockSpec((_TC_BLK, D), lambda i: (i, 0)),
            pl.BlockSpec((NC, _TC_BLK, DA), lambda i: (0, i, 0)),
        ] + _wspecs(),
        out_specs=[pl.BlockSpec((_TC_BLK, D), lambda i: (i, 0)),
                   pl.BlockSpec((_TC_BLK, _IW), lambda i: (i, 0))],
        out_shape=[jax.ShapeDtypeStruct((N, D), jnp.float32),
                   jax.ShapeDtypeStruct((N, _IW), jnp.float32)],
    )(h, s, *weights)


def _tc_dense2(h, s, inv, *weights):
    return pl.pallas_call(
        _tc2_body,
        grid=(N // _TC_BLK,),
        in_specs=[
            pl.BlockSpec((_TC_BLK, D), lambda i: (i, 0)),
            pl.BlockSpec((NC, _TC_BLK, D), lambda i: (0, i, 0)),
            pl.BlockSpec((_TC_BLK, _IW), lambda i: (i, 0)),
        ] + _wspecs(),
        out_specs=pl.BlockSpec((_TC_BLK, D), lambda i: (i, 0)),
        out_shape=jax.ShapeDtypeStruct((N, D), jnp.float32),
    )(h, s, inv, *weights)


def kernel(x, edge_index, W_self, W_nbr, b_gnn, W1, b1, W2, b2,
           ln1_g, ln1_b, ln2_g, ln2_b):
    src = edge_index[0].reshape(NC, NS, NBL, NB, K)
    dst = edge_index[1].reshape(NC, NS, NBL, NB, K)
    zs_aug = jnp.zeros((RS, DA), jnp.float32)
    zs = jnp.zeros((RS, D), jnp.float32)
    x_aug = jnp.concatenate([x, jnp.ones((N, CW), jnp.float32)], axis=1)

    r2 = lambda a: a.reshape(1, -1)
    w = lambda i: (W_self[i], W_nbr[i], r2(b_gnn[i]), W1[i], r2(b1[i]),
                   W2[i], r2(b2[i]), r2(ln1_g[i]), r2(ln1_b[i]),
                   r2(ln2_g[i]), r2(ln2_b[i]))

    s_parts = _get_sc_kernel(DA)(x_aug, src, dst, zs_aug)
    h, inv = _tc_dense1(x, s_parts, *w(0))
    s_parts = _get_sc_kernel(D)(h, src, dst, zs)
    h = _tc_dense2(h, s_parts, inv, *w(1))
    return h
